# baseline (speedup 1.0000x reference)
import functools

import jax
import jax.numpy as jnp
from jax import lax
from jax.experimental import pallas as pl
from jax.experimental.pallas import tpu as pltpu, tpu_sc as plsc

_NC = 2
_NS = 16
_NW = _NC * _NS
_L = 16

_ROWS = 4096
_COLS = 200
_ROWS_PER_TILE = _ROWS // _NW
_FULL_WIN = _COLS // _L
_TAIL_OFF = _COLS - _L
_TABLE = 256


def _build_sc_call():
    mesh = plsc.VectorSubcoreMesh(core_axis_name="c", subcore_axis_name="s")

    @functools.partial(
        pl.kernel,
        out_type=jax.ShapeDtypeStruct((_ROWS, _COLS), jnp.int32),
        mesh=mesh,
        scratch_types=[
            pltpu.VMEM((_TABLE,), jnp.int32),
            pltpu.VMEM((_ROWS_PER_TILE, _COLS), jnp.int32),
            pltpu.VMEM((_ROWS_PER_TILE, _COLS), jnp.int32),
        ],
        compiler_params=pltpu.CompilerParams(needs_layout_passes=False, use_tc_tiling_on_sc=True),
    )
    def lookup(inp_hbm, tab_hbm, out_hbm, tab_v, inp_v, out_v):
        wid = lax.axis_index("s") * _NC + lax.axis_index("c")
        r0 = wid * _ROWS_PER_TILE
        pltpu.sync_copy(tab_hbm, tab_v)
        pltpu.sync_copy(inp_hbm.at[pl.ds(r0, _ROWS_PER_TILE), :], inp_v)

        @plsc.parallel_loop(0, _ROWS_PER_TILE, unroll=2)
        def _(r):
            for w in range(_FULL_WIN):
                off = w * _L
                idx = inp_v[r, pl.ds(off, _L)]
                out_v[r, pl.ds(off, _L)] = plsc.load_gather(tab_v, [idx])
            idx = inp_v[r, pl.ds(_TAIL_OFF, _L)]
            out_v[r, pl.ds(_TAIL_OFF, _L)] = plsc.load_gather(tab_v, [idx])

        pltpu.sync_copy(out_v, out_hbm.at[pl.ds(r0, _ROWS_PER_TILE), :])

    return lookup


_lookup = _build_sc_call()


@jax.jit
def kernel(inputs, table):
    return _lookup(inputs, table)


# speedup vs baseline: 1.3574x; 1.3574x over previous
import functools

import jax
import jax.numpy as jnp
from jax import lax
from jax.experimental import pallas as pl
from jax.experimental.pallas import tpu as pltpu, tpu_sc as plsc

_NC = 2
_NS = 16
_NW = _NC * _NS
_L = 16

_ROWS = 200
_COLS = 4096
_COLS_PER_TILE = _COLS // _NW
_WIN = _COLS_PER_TILE // _L
_TABLE = 256


def _build_sc_call():
    mesh = plsc.VectorSubcoreMesh(core_axis_name="c", subcore_axis_name="s")

    @functools.partial(
        pl.kernel,
        out_type=jax.ShapeDtypeStruct((_ROWS, _COLS), jnp.int32),
        mesh=mesh,
        scratch_types=[
            pltpu.VMEM((_TABLE,), jnp.int32),
            pltpu.VMEM((_ROWS, _COLS_PER_TILE), jnp.int32),
            pltpu.VMEM((_ROWS, _COLS_PER_TILE), jnp.int32),
        ],
        compiler_params=pltpu.CompilerParams(
            needs_layout_passes=False, use_tc_tiling_on_sc=True
        ),
    )
    def lookup(inp_hbm, tab_hbm, out_hbm, tab_v, inp_v, out_v):
        wid = lax.axis_index("s") * _NC + lax.axis_index("c")
        c0 = wid * _COLS_PER_TILE
        pltpu.sync_copy(tab_hbm, tab_v)
        pltpu.sync_copy(inp_hbm.at[:, pl.ds(c0, _COLS_PER_TILE)], inp_v)

        @plsc.parallel_loop(0, _ROWS, unroll=2)
        def _(r):
            for w in range(_WIN):
                off = w * _L
                idx = inp_v[r, pl.ds(off, _L)]
                out_v[r, pl.ds(off, _L)] = plsc.load_gather(tab_v, [idx])

        pltpu.sync_copy(out_v, out_hbm.at[:, pl.ds(c0, _COLS_PER_TILE)])

    return lookup


_lookup = _build_sc_call()


@jax.jit
def kernel(inputs, table):
    return _lookup(inputs.T, table).T
